# baseline (speedup 1.0000x reference)
import functools

import jax
import jax.numpy as jnp
from jax import lax
from jax.experimental import pallas as pl
from jax.experimental.pallas import tpu as pltpu
from jax.experimental.pallas import tpu_sc as plsc

BATCH = 16384
DIM = 64
LANES = 16
NUM_CORES = 2
NUM_SUBCORES = 16
NW = NUM_CORES * NUM_SUBCORES
BPW = BATCH // NW
CHUNK = 128
NCHUNK = BPW // CHUNK

_mesh = plsc.VectorSubcoreMesh(core_axis_name="c", subcore_axis_name="s")


@functools.partial(
    pl.kernel,
    mesh=_mesh,
    compiler_params=pltpu.CompilerParams(use_tc_tiling_on_sc=False),
    out_type=jax.ShapeDtypeStruct((BATCH, DIM), jnp.float32),
    scratch_types=[
        pltpu.VMEM((NCHUNK, CHUNK), jnp.int32),
        pltpu.VMEM((NCHUNK, CHUNK), jnp.int32),
        pltpu.VMEM((BPW, DIM), jnp.float32),
        pltpu.VMEM((BPW, DIM), jnp.float32),
        pltpu.SemaphoreType.DMA,
        pltpu.SemaphoreType.DMA,
    ],
)
def _gmf_sc(user_hbm, item_hbm, utab_hbm, itab_hbm, out_hbm,
            uidx_v, iidx_v, urows_v, irows_v, sem_u, sem_i):
    wid = lax.axis_index("s") * NUM_CORES + lax.axis_index("c")
    base = wid * BPW

    pltpu.sync_copy(user_hbm.at[pl.ds(wid * NCHUNK, NCHUNK)], uidx_v)
    pltpu.sync_copy(item_hbm.at[pl.ds(wid * NCHUNK, NCHUNK)], iidx_v)

    for j in range(NCHUNK):
        pltpu.async_copy(utab_hbm.at[uidx_v.at[j]],
                         urows_v.at[pl.ds(j * CHUNK, CHUNK)], sem_u)
        pltpu.async_copy(itab_hbm.at[iidx_v.at[j]],
                         irows_v.at[pl.ds(j * CHUNK, CHUNK)], sem_i)
    for j in range(NCHUNK):
        pltpu.make_async_copy(utab_hbm.at[uidx_v.at[j]],
                              urows_v.at[pl.ds(j * CHUNK, CHUNK)], sem_u).wait()
        pltpu.make_async_copy(itab_hbm.at[iidx_v.at[j]],
                              irows_v.at[pl.ds(j * CHUNK, CHUNK)], sem_i).wait()

    def row_body(r, carry):
        for c in range(DIM // LANES):
            s = pl.ds(c * LANES, LANES)
            urows_v[r, s] = urows_v[r, s] * irows_v[r, s]
        return carry

    lax.fori_loop(0, BPW, row_body, 0)

    pltpu.sync_copy(urows_v, out_hbm.at[pl.ds(base, BPW)])


def kernel(user, item, user_table, item_table):
    user2 = user.reshape(NW * NCHUNK, CHUNK)
    item2 = item.reshape(NW * NCHUNK, CHUNK)
    return _gmf_sc(user2, item2, user_table, item_table)


# speedup vs baseline: 1.5369x; 1.5369x over previous
import functools

import jax
import jax.numpy as jnp
from jax import lax
from jax.experimental import pallas as pl
from jax.experimental.pallas import tpu as pltpu
from jax.experimental.pallas import tpu_sc as plsc

BATCH = 16384
DIM = 64
LANES = 16
NUM_CORES = 2
NUM_SUBCORES = 16
NW = NUM_CORES * NUM_SUBCORES
BPW = BATCH // NW
HPW = BPW // 2
CDMA = 16
NCH = BPW // CDMA

_mesh = plsc.VectorSubcoreMesh(core_axis_name="c", subcore_axis_name="s")


@functools.partial(
    pl.kernel,
    mesh=_mesh,
    out_type=jax.ShapeDtypeStruct((BATCH // 2, 2 * DIM), jnp.float32),
    scratch_types=[
        pltpu.VMEM((BPW,), jnp.int32),
        pltpu.VMEM((BPW,), jnp.int32),
        pltpu.VMEM((HPW, 2 * DIM), jnp.float32),
        pltpu.VMEM((HPW, 2 * DIM), jnp.float32),
        pltpu.SemaphoreType.DMA,
        pltpu.SemaphoreType.DMA,
    ],
)
def _gmf_sc(user_hbm, item_hbm, utab_hbm, itab_hbm, out_hbm,
            uidx_v, iidx_v, urows_v, irows_v, sem_u, sem_i):
    wid = lax.axis_index("s") * NUM_CORES + lax.axis_index("c")
    base = wid * BPW

    pltpu.sync_copy(user_hbm.at[pl.ds(base, BPW)], uidx_v)
    pltpu.sync_copy(item_hbm.at[pl.ds(base, BPW)], iidx_v)

    def chunk_body(c, carry):
        cb = c * CDMA
        ch = c * (CDMA // 2)
        uvec = uidx_v[pl.ds(cb, CDMA)]
        ivec = iidx_v[pl.ds(cb, CDMA)]
        for j in range(CDMA):
            dst_row = ch + j // 2
            dst_col = pl.ds((j % 2) * DIM, DIM)
            pltpu.async_copy(utab_hbm.at[uvec[j]],
                             urows_v.at[dst_row, dst_col], sem_u)
            pltpu.async_copy(itab_hbm.at[ivec[j]],
                             irows_v.at[dst_row, dst_col], sem_i)
        for j in range(CDMA):
            pltpu.make_async_copy(utab_hbm.at[0],
                                  urows_v.at[0, pl.ds(0, DIM)], sem_u).wait()
            pltpu.make_async_copy(itab_hbm.at[0],
                                  irows_v.at[0, pl.ds(0, DIM)], sem_i).wait()
        return carry

    lax.fori_loop(0, NCH, chunk_body, 0)

    def row_body(q, carry):
        for c in range(2 * DIM // LANES):
            s = pl.ds(c * LANES, LANES)
            urows_v[q, s] = urows_v[q, s] * irows_v[q, s]
        return carry

    lax.fori_loop(0, HPW, row_body, 0)

    pltpu.sync_copy(urows_v, out_hbm.at[pl.ds(wid * HPW, HPW)])


def kernel(user, item, user_table, item_table):
    out2 = _gmf_sc(user, item, user_table, item_table)
    return out2.reshape(BATCH, DIM)
